# Initial kernel scaffold; baseline (speedup 1.0000x reference)
import jax
import jax.numpy as jnp
from jax.experimental import pallas as pl


def kernel(weight):
    raise NotImplementedError("write your pallas kernel here")


# speedup vs baseline: 33.7762x; 33.7762x over previous
import jax
import jax.numpy as jnp
from jax import lax
from jax.experimental import pallas as pl
from jax.experimental.pallas import tpu as pltpu

N = 4096
NUM_W = N * (N - 1) // 2
WPAD = N + 128
RB = 128
K1_STEPS = N // RB
B2 = 512
G2 = N // B2


def _row_start(r):
    return r * (N - 1) - (r * (r - 1)) // 2 - r - 1


def _aligned_start(start):
    return jnp.clip(128 * (start // 128), 0, NUM_W - WPAD)


def _k1_body(w_ref, u_ref, sem):
    k = pl.program_id(0)

    def run(t, do_wait):
        r = k * RB + t
        start = _row_start(r)
        a = pl.multiple_of(_aligned_start(start), 128)

        @pl.when(r > 0)
        def _():
            cp = pltpu.make_async_copy(w_ref.at[pl.ds(a, WPAD)],
                                       u_ref.at[r], sem)
            if do_wait:
                cp.wait()
            else:
                cp.start()

        @pl.when(r == 0)
        def _():
            cp = pltpu.make_async_copy(w_ref.at[pl.ds(0, N)],
                                       u_ref.at[0, pl.ds(128, N)], sem)
            if do_wait:
                cp.wait()
            else:
                cp.start()

    def issue(t, c):
        run(t, False)
        return c

    def drain(t, c):
        run(t, True)
        return c

    lax.fori_loop(0, RB, issue, 0)
    lax.fori_loop(0, RB, drain, 0)


def _phi(rows):
    start = rows * (N - 1) - (rows * (rows - 1)) // 2 - rows - 1
    a = _aligned_start(start)
    return jnp.where(rows == 0, 127, start - a)


def _gather_block(main, nxt, phi):
    lane = lax.broadcasted_iota(jnp.int32, (B2, 128), 1)
    m = lane + phi
    idx = jnp.bitwise_and(m, 127)
    tiles = []
    for t in range(B2 // 128):
        t0 = main[:, 128 * t:128 * (t + 1)]
        t1 = main[:, 128 * (t + 1):128 * (t + 2)] if t < B2 // 128 - 1 else nxt
        g0 = jnp.take_along_axis(t0, idx, axis=1, mode="promise_in_bounds")
        g1 = jnp.take_along_axis(t1, idx, axis=1, mode="promise_in_bounds")
        tiles.append(jnp.where(m < 128, g0, g1))
    return jnp.concatenate(tiles, axis=1)


def _k2_body(a_main, a_next, b_main, b_next, o_ref):
    i = pl.program_id(0)
    j = pl.program_id(1)
    rows_i = i * B2 + lax.broadcasted_iota(jnp.int32, (B2, 1), 0)
    rows_j = j * B2 + lax.broadcasted_iota(jnp.int32, (B2, 1), 0)
    ub = _gather_block(a_main[...], a_next[...], _phi(rows_i))
    lb = _gather_block(b_main[...], b_next[...], _phi(rows_j))
    gr = i * B2 + lax.broadcasted_iota(jnp.int32, (B2, B2), 0)
    gc = j * B2 + lax.broadcasted_iota(jnp.int32, (B2, B2), 1)
    zero = jnp.zeros((), jnp.float32)
    o_ref[...] = (jnp.where(gc > gr, ub, zero)
                  - jnp.where(gc < gr, lb.T, zero))


def kernel(weight):
    w = weight.reshape(NUM_W)

    u2 = pl.pallas_call(
        _k1_body,
        grid=(K1_STEPS,),
        in_specs=[pl.BlockSpec(memory_space=pl.ANY)],
        out_specs=pl.BlockSpec(memory_space=pl.ANY),
        out_shape=jax.ShapeDtypeStruct((N, WPAD), jnp.float32),
        scratch_shapes=[pltpu.SemaphoreType.DMA],
    )(w)

    out = pl.pallas_call(
        _k2_body,
        grid=(G2, G2),
        in_specs=[
            pl.BlockSpec((B2, B2), lambda i, j: (i, j)),
            pl.BlockSpec((B2, 128), lambda i, j: (i, 4 * j + 4)),
            pl.BlockSpec((B2, B2), lambda i, j: (j, i)),
            pl.BlockSpec((B2, 128), lambda i, j: (j, 4 * i + 4)),
        ],
        out_specs=pl.BlockSpec((B2, B2), lambda i, j: (i, j)),
        out_shape=jax.ShapeDtypeStruct((N, N), jnp.float32),
    )(u2, u2, u2, u2)
    return out
